# baseline (speedup 1.0000x reference)
import functools

import jax
import jax.numpy as jnp
from jax import lax
from jax.experimental import pallas as pl
from jax.experimental.pallas import tpu as pltpu
from jax.experimental.pallas import tpu_sc as plsc

B = 16384
NW = 32
BPW = B // NW

MF_D = 8
MLP_D = 32


def _sc_gather(user2d, item2d, mfu_t, mfi_t, mlpu_t, mlpi_t):
    mesh = plsc.VectorSubcoreMesh(core_axis_name="c", subcore_axis_name="s")

    @functools.partial(
        pl.kernel,
        mesh=mesh,
        compiler_params=pltpu.CompilerParams(needs_layout_passes=False),
        out_type=[
            jax.ShapeDtypeStruct((B * MF_D,), jnp.float32),
            jax.ShapeDtypeStruct((B * MF_D,), jnp.float32),
            jax.ShapeDtypeStruct((B * MLP_D,), jnp.float32),
            jax.ShapeDtypeStruct((B * MLP_D,), jnp.float32),
        ],
        scratch_types=[
            pltpu.VMEM((BPW,), jnp.int32),
            pltpu.VMEM((BPW,), jnp.int32),
            pltpu.VMEM((BPW * MF_D,), jnp.float32),
            pltpu.VMEM((BPW * MF_D,), jnp.float32),
            pltpu.VMEM((BPW * MLP_D,), jnp.float32),
            pltpu.VMEM((BPW * MLP_D,), jnp.float32),
            pltpu.SemaphoreType.DMA,
        ],
    )
    def k(u_hbm, i_hbm, mfu_tr, mfi_tr, mlpu_tr, mlpi_tr,
          mfu_o, mfi_o, mlpu_o, mlpi_o,
          uidx, iidx, mfu_v, mfi_v, mlpu_v, mlpi_v, sem):
        wid = lax.axis_index("s") * 2 + lax.axis_index("c")
        pltpu.sync_copy(u_hbm.at[wid], uidx)
        pltpu.sync_copy(i_hbm.at[wid], iidx)
        lane = lax.iota(jnp.int32, 16)

        def body(v, _):
            uvec = uidx[pl.ds(v * 16, 16)]
            ivec = iidx[pl.ds(v * 16, 16)]
            umf = uvec * MF_D
            imf = ivec * MF_D
            umlp = uvec * MLP_D
            imlp = ivec * MLP_D
            for l in range(16):
                a = pl.multiple_of(jnp.sum(jnp.where(lane == l, umf, 0)), MF_D)
                b = pl.multiple_of(jnp.sum(jnp.where(lane == l, imf, 0)), MF_D)
                c = pl.multiple_of(
                    jnp.sum(jnp.where(lane == l, umlp, 0)), MLP_D)
                d = pl.multiple_of(
                    jnp.sum(jnp.where(lane == l, imlp, 0)), MLP_D)
                s = v * 16 + l
                pltpu.async_copy(
                    mfu_tr.at[pl.ds(a, MF_D)],
                    mfu_v.at[pl.ds(s * MF_D, MF_D)], sem)
                pltpu.async_copy(
                    mlpu_tr.at[pl.ds(c, MLP_D)],
                    mlpu_v.at[pl.ds(s * MLP_D, MLP_D)], sem)
                pltpu.async_copy(
                    mfi_tr.at[pl.ds(b, MF_D)],
                    mfi_v.at[pl.ds(s * MF_D, MF_D)], sem)
                pltpu.async_copy(
                    mlpi_tr.at[pl.ds(d, MLP_D)],
                    mlpi_v.at[pl.ds(s * MLP_D, MLP_D)], sem)
            return ()

        lax.fori_loop(0, BPW // 16, body, (), unroll=False)
        pltpu.make_async_copy(
            mfu_o.at[pl.ds(0, BPW * MF_D)], mfu_v, sem).wait()
        pltpu.make_async_copy(
            mfi_o.at[pl.ds(0, BPW * MF_D)], mfi_v, sem).wait()
        pltpu.make_async_copy(
            mlpu_o.at[pl.ds(0, BPW * MLP_D)], mlpu_v, sem).wait()
        pltpu.make_async_copy(
            mlpi_o.at[pl.ds(0, BPW * MLP_D)], mlpi_v, sem).wait()
        pltpu.sync_copy(mfu_v, mfu_o.at[pl.ds(wid * BPW * MF_D, BPW * MF_D)])
        pltpu.sync_copy(mfi_v, mfi_o.at[pl.ds(wid * BPW * MF_D, BPW * MF_D)])
        pltpu.sync_copy(
            mlpu_v, mlpu_o.at[pl.ds(wid * BPW * MLP_D, BPW * MLP_D)])
        pltpu.sync_copy(
            mlpi_v, mlpi_o.at[pl.ds(wid * BPW * MLP_D, BPW * MLP_D)])

    return k(user2d, item2d, mfu_t, mfi_t, mlpu_t, mlpi_t)


def _tc_body(mfu_r, mfi_r, mlpu_r, mlpi_r,
             w0_r, b0_r, w1_r, b1_r, w2_r, b2_r, wp_r, bp_r, o_r):
    w0 = w0_r[...]
    h = jnp.dot(mlpu_r[...], w0[:MLP_D, :], preferred_element_type=jnp.float32)
    h = h + jnp.dot(mlpi_r[...], w0[MLP_D:, :], preferred_element_type=jnp.float32)
    h = jnp.maximum(h + b0_r[...], 0.0)
    h = jnp.maximum(
        jnp.dot(h, w1_r[...], preferred_element_type=jnp.float32) + b1_r[...], 0.0)
    h = jnp.maximum(
        jnp.dot(h, w2_r[...], preferred_element_type=jnp.float32) + b2_r[...], 0.0)
    wp = wp_r[...]
    p = jnp.dot(mfu_r[...] * mfi_r[...], wp[:MF_D, :],
                preferred_element_type=jnp.float32)
    p = p + jnp.dot(h, wp[MF_D:, :], preferred_element_type=jnp.float32)
    o_r[...] = p + bp_r[...]


def _tc_mlp(mfu, mfi, mlpu, mlpi, W0, b0, W1, b1, W2, b2, Wp, bp):
    BLK = 2048
    grid = (B // BLK,)

    def full(shape):
        return pl.BlockSpec(shape, lambda i: (0,) * len(shape))

    return pl.pallas_call(
        _tc_body,
        grid=grid,
        in_specs=[
            pl.BlockSpec((BLK, MF_D), lambda i: (i, 0)),
            pl.BlockSpec((BLK, MF_D), lambda i: (i, 0)),
            pl.BlockSpec((BLK, MLP_D), lambda i: (i, 0)),
            pl.BlockSpec((BLK, MLP_D), lambda i: (i, 0)),
            full(W0.shape), full(b0.shape), full(W1.shape), full(b1.shape),
            full(W2.shape), full(b2.shape), full(Wp.shape), full(bp.shape),
        ],
        out_specs=pl.BlockSpec((BLK, 1), lambda i: (i, 0)),
        out_shape=jax.ShapeDtypeStruct((B, 1), jnp.float32),
    )(mfu, mfi, mlpu, mlpi, W0, b0, W1, b1, W2, b2, Wp, bp)


def kernel(user, item, mf_emb_user, mf_emb_item, mlp_emb_user, mlp_emb_item,
           W0, b0, W1, b1, W2, b2, Wp, bp):
    user = user.astype(jnp.int32)
    item = item.astype(jnp.int32)
    u2 = user.reshape(NW, BPW)
    i2 = item.reshape(NW, BPW)
    mfu, mfi, mlpu, mlpi = _sc_gather(
        u2, i2,
        mf_emb_user.reshape(-1), mf_emb_item.reshape(-1),
        mlp_emb_user.reshape(-1), mlp_emb_item.reshape(-1))
    mfu = mfu.reshape(B, MF_D)
    mfi = mfi.reshape(B, MF_D)
    mlpu = mlpu.reshape(B, MLP_D)
    mlpi = mlpi.reshape(B, MLP_D)
    return _tc_mlp(
        mfu, mfi, mlpu, mlpi,
        W0, b0.reshape(1, -1), W1, b1.reshape(1, -1),
        W2, b2.reshape(1, -1), Wp, bp.reshape(1, 1))


# speedup vs baseline: 1.1834x; 1.1834x over previous
import functools

import jax
import jax.numpy as jnp
from jax import lax
from jax.experimental import pallas as pl
from jax.experimental.pallas import tpu as pltpu
from jax.experimental.pallas import tpu_sc as plsc

B = 16384
NW = 32
BPW = B // NW

MF_D = 8
W = 32
PACK_R = BPW * W // 128
OUT_R = B * W // 128


def _sc_gather(idx2d, tab, shift):
    mesh = plsc.VectorSubcoreMesh(core_axis_name="c", subcore_axis_name="s")

    @functools.partial(
        pl.kernel,
        mesh=mesh,
        out_type=[jax.ShapeDtypeStruct((OUT_R, 128), jnp.float32)],
        scratch_types=[
            pltpu.VMEM((BPW,), jnp.int32),
            pltpu.VMEM((BPW, W), jnp.float32),
            pltpu.VMEM((PACK_R, 128), jnp.float32),
            pltpu.SemaphoreType.DMA,
        ],
    )
    def k(x_hbm, x_tr, x_o, xidx, x_v, pack, sem):
        wid = lax.axis_index("s") * 2 + lax.axis_index("c")
        pltpu.sync_copy(x_hbm.at[wid], xidx)

        def body(v, _):
            xvec = xidx[pl.ds(v * 16, 16)]
            for l in range(16):
                s = v * 16 + l
                x = xvec[l]
                if shift:
                    x = lax.shift_right_logical(x, shift)
                pltpu.async_copy(
                    x_tr.at[pl.ds(x, 1)], x_v.at[pl.ds(s, 1)], sem)
            return ()

        lax.fori_loop(0, BPW // 16, body, (), unroll=False)
        pltpu.make_async_copy(x_tr.at[pl.ds(0, BPW)], x_v, sem).wait()

        def prow(r, _):
            for j in range(4):
                for h in range(2):
                    pack[r, pl.ds(j * W + h * 16, 16)] = (
                        x_v[4 * r + j, pl.ds(h * 16, 16)])
            return ()

        lax.fori_loop(0, PACK_R, prow, (), unroll=False)
        pltpu.sync_copy(pack, x_o.at[pl.ds(wid * PACK_R, PACK_R)])

    return k(idx2d, tab)


def _quad_select(x32, b0, b1):
    lo = jnp.where(b0 > 0, x32[:, MF_D:2 * MF_D], x32[:, :MF_D])
    hi = jnp.where(b0 > 0, x32[:, 3 * MF_D:], x32[:, 2 * MF_D:3 * MF_D])
    return jnp.where(b1 > 0, hi, lo)


def _tc_body(mfu_r, mfi_r, mlpu_r, mlpi_r, ub0_r, ub1_r, ib0_r, ib1_r,
             w0_r, b0_r, w1_r, b1_r, w2_r, b2_r, wp_r, bp_r, o_r):
    w0 = w0_r[...]
    h = jnp.dot(mlpu_r[...], w0[:W, :], preferred_element_type=jnp.float32)
    h = h + jnp.dot(mlpi_r[...], w0[W:, :], preferred_element_type=jnp.float32)
    h = jnp.maximum(h + b0_r[...], 0.0)
    h = jnp.maximum(
        jnp.dot(h, w1_r[...], preferred_element_type=jnp.float32) + b1_r[...], 0.0)
    h = jnp.maximum(
        jnp.dot(h, w2_r[...], preferred_element_type=jnp.float32) + b2_r[...], 0.0)
    mfu = _quad_select(mfu_r[...], ub0_r[...], ub1_r[...])
    mfi = _quad_select(mfi_r[...], ib0_r[...], ib1_r[...])
    wp = wp_r[...]
    p = jnp.dot(mfu * mfi, wp[:MF_D, :], preferred_element_type=jnp.float32)
    p = p + jnp.dot(h, wp[MF_D:, :], preferred_element_type=jnp.float32)
    o_r[...] = p + bp_r[...]


def _tc_mlp(mfu, mfi, mlpu, mlpi, ub0, ub1, ib0, ib1,
            W0, b0, W1, b1, W2, b2, Wp, bp):
    BLK = 2048
    grid = (B // BLK,)

    def full(shape):
        return pl.BlockSpec(shape, lambda i: (0,) * len(shape))

    return pl.pallas_call(
        _tc_body,
        grid=grid,
        in_specs=[
            pl.BlockSpec((BLK, W), lambda i: (i, 0)),
            pl.BlockSpec((BLK, W), lambda i: (i, 0)),
            pl.BlockSpec((BLK, W), lambda i: (i, 0)),
            pl.BlockSpec((BLK, W), lambda i: (i, 0)),
            pl.BlockSpec((BLK, 1), lambda i: (i, 0)),
            pl.BlockSpec((BLK, 1), lambda i: (i, 0)),
            pl.BlockSpec((BLK, 1), lambda i: (i, 0)),
            pl.BlockSpec((BLK, 1), lambda i: (i, 0)),
            full(W0.shape), full(b0.shape), full(W1.shape), full(b1.shape),
            full(W2.shape), full(b2.shape), full(Wp.shape), full(bp.shape),
        ],
        out_specs=pl.BlockSpec((BLK, 1), lambda i: (i, 0)),
        out_shape=jax.ShapeDtypeStruct((B, 1), jnp.float32),
    )(mfu, mfi, mlpu, mlpi, ub0, ub1, ib0, ib1,
      W0, b0, W1, b1, W2, b2, Wp, bp)


def kernel(user, item, mf_emb_user, mf_emb_item, mlp_emb_user, mlp_emb_item,
           W0, b0, W1, b1, W2, b2, Wp, bp):
    user = user.astype(jnp.int32)
    item = item.astype(jnp.int32)
    u2 = user.reshape(NW, BPW)
    i2 = item.reshape(NW, BPW)
    (mfu,) = _sc_gather(u2, mf_emb_user.reshape(-1, W), 2)
    (mfi,) = _sc_gather(i2, mf_emb_item.reshape(-1, W), 2)
    (mlpu,) = _sc_gather(u2, mlp_emb_user, 0)
    (mlpi,) = _sc_gather(i2, mlp_emb_item, 0)
    mfu = mfu.reshape(B, W)
    mfi = mfi.reshape(B, W)
    mlpu = mlpu.reshape(B, W)
    mlpi = mlpi.reshape(B, W)
    ub0 = (user & 1).reshape(B, 1)
    ub1 = (user & 2).reshape(B, 1)
    ib0 = (item & 1).reshape(B, 1)
    ib1 = (item & 2).reshape(B, 1)
    return _tc_mlp(
        mfu, mfi, mlpu, mlpi, ub0, ub1, ib0, ib1,
        W0, b0.reshape(1, -1), W1, b1.reshape(1, -1),
        W2, b2.reshape(1, -1), Wp, bp.reshape(1, 1))


# speedup vs baseline: 1.1842x; 1.0007x over previous
import functools

import jax
import jax.numpy as jnp
from jax import lax
from jax.experimental import pallas as pl
from jax.experimental.pallas import tpu as pltpu
from jax.experimental.pallas import tpu_sc as plsc

B = 16384
NW = 32
BPW = B // NW

MF_D = 8
W = 32
PACK_R = BPW * W // 128
OUT_R = B * W // 128


def _sc_gather(idx2d, tab, shift):
    mesh = plsc.VectorSubcoreMesh(core_axis_name="c", subcore_axis_name="s")

    @functools.partial(
        pl.kernel,
        mesh=mesh,
        compiler_params=pltpu.CompilerParams(has_side_effects=False),
        out_type=[jax.ShapeDtypeStruct((OUT_R, 128), jnp.float32)],
        scratch_types=[
            pltpu.VMEM((BPW,), jnp.int32),
            pltpu.VMEM((BPW, W), jnp.float32),
            pltpu.VMEM((PACK_R, 128), jnp.float32),
            pltpu.SemaphoreType.DMA,
        ],
    )
    def k(x_hbm, x_tr, x_o, xidx, x_v, pack, sem):
        wid = lax.axis_index("s") * 2 + lax.axis_index("c")
        pltpu.sync_copy(x_hbm.at[wid], xidx)

        def body(v, _):
            xvec = xidx[pl.ds(v * 16, 16)]
            for l in range(16):
                s = v * 16 + l
                x = xvec[l]
                if shift:
                    x = lax.shift_right_logical(x, shift)
                pltpu.async_copy(
                    x_tr.at[pl.ds(x, 1)], x_v.at[pl.ds(s, 1)], sem)
            return ()

        lax.fori_loop(0, BPW // 16, body, (), unroll=False)
        pltpu.make_async_copy(x_tr.at[pl.ds(0, BPW)], x_v, sem).wait()

        def prow(r, _):
            for j in range(4):
                for h in range(2):
                    pack[r, pl.ds(j * W + h * 16, 16)] = (
                        x_v[4 * r + j, pl.ds(h * 16, 16)])
            return ()

        lax.fori_loop(0, PACK_R, prow, (), unroll=False)
        pltpu.sync_copy(pack, x_o.at[pl.ds(wid * PACK_R, PACK_R)])

    return k(idx2d, tab)


def _quad_select(x32, b0, b1):
    lo = jnp.where(b0 > 0, x32[:, MF_D:2 * MF_D], x32[:, :MF_D])
    hi = jnp.where(b0 > 0, x32[:, 3 * MF_D:], x32[:, 2 * MF_D:3 * MF_D])
    return jnp.where(b1 > 0, hi, lo)


def _tc_body(mfu_r, mfi_r, mlpu_r, mlpi_r, ub0_r, ub1_r, ib0_r, ib1_r,
             w0_r, b0_r, w1_r, b1_r, w2_r, b2_r, wp_r, bp_r, o_r):
    w0 = w0_r[...]
    h = jnp.dot(mlpu_r[...], w0[:W, :], preferred_element_type=jnp.float32)
    h = h + jnp.dot(mlpi_r[...], w0[W:, :], preferred_element_type=jnp.float32)
    h = jnp.maximum(h + b0_r[...], 0.0)
    h = jnp.maximum(
        jnp.dot(h, w1_r[...], preferred_element_type=jnp.float32) + b1_r[...], 0.0)
    h = jnp.maximum(
        jnp.dot(h, w2_r[...], preferred_element_type=jnp.float32) + b2_r[...], 0.0)
    mfu = _quad_select(mfu_r[...], ub0_r[...], ub1_r[...])
    mfi = _quad_select(mfi_r[...], ib0_r[...], ib1_r[...])
    wp = wp_r[...]
    p = jnp.dot(mfu * mfi, wp[:MF_D, :], preferred_element_type=jnp.float32)
    p = p + jnp.dot(h, wp[MF_D:, :], preferred_element_type=jnp.float32)
    o_r[...] = p + bp_r[...]


def _tc_mlp(mfu, mfi, mlpu, mlpi, ub0, ub1, ib0, ib1,
            W0, b0, W1, b1, W2, b2, Wp, bp):
    BLK = 2048
    grid = (B // BLK,)

    def full(shape):
        return pl.BlockSpec(shape, lambda i: (0,) * len(shape))

    return pl.pallas_call(
        _tc_body,
        grid=grid,
        in_specs=[
            pl.BlockSpec((BLK, W), lambda i: (i, 0)),
            pl.BlockSpec((BLK, W), lambda i: (i, 0)),
            pl.BlockSpec((BLK, W), lambda i: (i, 0)),
            pl.BlockSpec((BLK, W), lambda i: (i, 0)),
            pl.BlockSpec((BLK, 1), lambda i: (i, 0)),
            pl.BlockSpec((BLK, 1), lambda i: (i, 0)),
            pl.BlockSpec((BLK, 1), lambda i: (i, 0)),
            pl.BlockSpec((BLK, 1), lambda i: (i, 0)),
            full(W0.shape), full(b0.shape), full(W1.shape), full(b1.shape),
            full(W2.shape), full(b2.shape), full(Wp.shape), full(bp.shape),
        ],
        out_specs=pl.BlockSpec((BLK, 1), lambda i: (i, 0)),
        out_shape=jax.ShapeDtypeStruct((B, 1), jnp.float32),
    )(mfu, mfi, mlpu, mlpi, ub0, ub1, ib0, ib1,
      W0, b0, W1, b1, W2, b2, Wp, bp)


def kernel(user, item, mf_emb_user, mf_emb_item, mlp_emb_user, mlp_emb_item,
           W0, b0, W1, b1, W2, b2, Wp, bp):
    user = user.astype(jnp.int32)
    item = item.astype(jnp.int32)
    u2 = user.reshape(NW, BPW)
    i2 = item.reshape(NW, BPW)
    (mfu,) = _sc_gather(u2, mf_emb_user.reshape(-1, W), 2)
    (mfi,) = _sc_gather(i2, mf_emb_item.reshape(-1, W), 2)
    (mlpu,) = _sc_gather(u2, mlp_emb_user, 0)
    (mlpi,) = _sc_gather(i2, mlp_emb_item, 0)
    mfu = mfu.reshape(B, W)
    mfi = mfi.reshape(B, W)
    mlpu = mlpu.reshape(B, W)
    mlpi = mlpi.reshape(B, W)
    ub0 = (user & 1).reshape(B, 1)
    ub1 = (user & 2).reshape(B, 1)
    ib0 = (item & 1).reshape(B, 1)
    ib1 = (item & 2).reshape(B, 1)
    return _tc_mlp(
        mfu, mfi, mlpu, mlpi, ub0, ub1, ib0, ib1,
        W0, b0.reshape(1, -1), W1, b1.reshape(1, -1),
        W2, b2.reshape(1, -1), Wp, bp.reshape(1, 1))


# speedup vs baseline: 1.1998x; 1.0132x over previous
import functools

import jax
import jax.numpy as jnp
from jax import lax
from jax.experimental import pallas as pl
from jax.experimental.pallas import tpu as pltpu
from jax.experimental.pallas import tpu_sc as plsc

B = 16384
NW = 32
BPW = B // NW

MF_D = 8
W = 32

MF_PR = BPW * MF_D // 128
MF_OR = B * MF_D // 128
MLP_PR = BPW * 16 // 128
MLP_OR = B * 16 // 128


def _sc_gather(user2d, item2d, mfu_t, mfi_t, mlpu_t, mlpi_t):
    mesh = plsc.VectorSubcoreMesh(core_axis_name="c", subcore_axis_name="s")

    @functools.partial(
        pl.kernel,
        mesh=mesh,
        compiler_params=pltpu.CompilerParams(needs_layout_passes=False),
        out_type=[
            jax.ShapeDtypeStruct((MF_OR, 128), jnp.float32),
            jax.ShapeDtypeStruct((MLP_OR, 128), jnp.int32),
            jax.ShapeDtypeStruct((MLP_OR, 128), jnp.int32),
        ],
        scratch_types=[
            pltpu.VMEM((BPW,), jnp.int32),
            pltpu.VMEM((BPW,), jnp.int32),
            pltpu.VMEM((BPW // 4, W), jnp.float32),
            pltpu.VMEM((BPW // 4, W), jnp.float32),
            pltpu.VMEM((BPW // 4, W), jnp.float32),
            pltpu.VMEM((BPW // 4, W), jnp.float32),
            pltpu.VMEM((MF_PR // 4, 128), jnp.float32),
            pltpu.VMEM((MLP_PR // 4, 128), jnp.int32),
            pltpu.SemaphoreType.DMA,
        ],
    )
    def k(u_hbm, i_hbm, mfu_tr, mfi_tr, mlpu_tr, mlpi_tr,
          mf_o, mlpu_o, mlpi_o,
          uidx, iidx, mfu_v, mfi_v, mlpu_v, mlpi_v,
          mf_p, mlp_p, sem):
        wid = lax.axis_index("s") * 2 + lax.axis_index("c")
        pltpu.sync_copy(u_hbm.at[wid], uidx)
        pltpu.sync_copy(i_hbm.at[wid], iidx)
        lane = lax.iota(jnp.int32, 16)
        lane8 = lane & 7
        half = lax.shift_right_logical(lane, 3)
        HC = BPW // 4
        MFH = MF_PR // 4
        MLH = MLP_PR // 4

        for hh in range(4):
            def body(v, _):
                uvec = uidx[pl.ds(hh * HC + v * 16, 16)]
                ivec = iidx[pl.ds(hh * HC + v * 16, 16)]
                for l in range(16):
                    s = v * 16 + l
                    u = uvec[l]
                    i = ivec[l]
                    uq = lax.shift_right_logical(u, 2)
                    iq = lax.shift_right_logical(i, 2)
                    pltpu.async_copy(
                        mfu_tr.at[pl.ds(uq, 1)], mfu_v.at[pl.ds(s, 1)], sem)
                    pltpu.async_copy(
                        mlpu_tr.at[pl.ds(u, 1)], mlpu_v.at[pl.ds(s, 1)], sem)
                    pltpu.async_copy(
                        mfi_tr.at[pl.ds(iq, 1)], mfi_v.at[pl.ds(s, 1)], sem)
                    pltpu.async_copy(
                        mlpi_tr.at[pl.ds(i, 1)], mlpi_v.at[pl.ds(s, 1)], sem)
                return ()

            lax.fori_loop(0, HC // 16, body, (), unroll=False)
            pltpu.make_async_copy(mfu_tr.at[pl.ds(0, HC)], mfu_v, sem).wait()
            pltpu.make_async_copy(mfi_tr.at[pl.ds(0, HC)], mfi_v, sem).wait()
            pltpu.make_async_copy(mlpu_tr.at[pl.ds(0, HC)], mlpu_v, sem).wait()
            pltpu.make_async_copy(mlpi_tr.at[pl.ds(0, HC)], mlpi_v, sem).wait()

            def mfrow(r, _):
                for q in range(8):
                    p = r * 8 + q
                    rowv = 2 * p + half
                    gidx = hh * HC + rowv
                    ucolv = (plsc.load_gather(uidx, [gidx]) & 3) * MF_D + lane8
                    icolv = (plsc.load_gather(iidx, [gidx]) & 3) * MF_D + lane8
                    uvals = plsc.load_gather(mfu_v, [rowv, ucolv])
                    ivals = plsc.load_gather(mfi_v, [rowv, icolv])
                    mf_p[r, pl.ds(q * 16, 16)] = uvals * ivals
                return ()

            lax.fori_loop(0, MFH, mfrow, (), unroll=False)
            pltpu.sync_copy(
                mf_p, mf_o.at[pl.ds(wid * MF_PR + hh * MFH, MFH)])

            def mlp_pack(buf, dst):
                def prow(r, _):
                    for q in range(8):
                        s = r * 8 + q
                        a = buf[s, pl.ds(0, 16)]
                        b = buf[s, pl.ds(16, 16)]
                        w = plsc.bitcast(
                            plsc.pack(
                                a, b, format=plsc.PackFormat.INTERLEAVED),
                            jnp.int32)
                        dst[r, pl.ds(q * 16, 16)] = w
                    return ()
                lax.fori_loop(0, MLH, prow, (), unroll=False)

            mlp_pack(mlpu_v, mlp_p)
            pltpu.sync_copy(
                mlp_p, mlpu_o.at[pl.ds(wid * MLP_PR + hh * MLH, MLH)])
            mlp_pack(mlpi_v, mlp_p)
            pltpu.sync_copy(
                mlp_p, mlpi_o.at[pl.ds(wid * MLP_PR + hh * MLH, MLH)])

    return k(user2d, item2d, mfu_t, mfi_t, mlpu_t, mlpi_t)


def _unpack2(w):
    a = lax.bitcast_convert_type(lax.shift_left(w, 16), jnp.float32)
    b = lax.bitcast_convert_type(w & jnp.int32(-65536), jnp.float32)
    return a, b


def _tc_body(mf_r, mlpu_r, mlpi_r,
             w0_r, b0_r, w1_r, b1_r, w2_r, b2_r, wp_r, bp_r, o_r):
    w0 = w0_r[...]
    ua, ub = _unpack2(mlpu_r[...])
    ia, ib = _unpack2(mlpi_r[...])
    h = jnp.dot(ua, w0[0:16, :], preferred_element_type=jnp.float32)
    h = h + jnp.dot(ub, w0[16:32, :], preferred_element_type=jnp.float32)
    h = h + jnp.dot(ia, w0[32:48, :], preferred_element_type=jnp.float32)
    h = h + jnp.dot(ib, w0[48:64, :], preferred_element_type=jnp.float32)
    h = jnp.maximum(h + b0_r[...], 0.0)
    h = jnp.maximum(
        jnp.dot(h, w1_r[...], preferred_element_type=jnp.float32) + b1_r[...], 0.0)
    h = jnp.maximum(
        jnp.dot(h, w2_r[...], preferred_element_type=jnp.float32) + b2_r[...], 0.0)
    wp = wp_r[...]
    p = jnp.dot(mf_r[...], wp[:MF_D, :],
                preferred_element_type=jnp.float32)
    p = p + jnp.dot(h, wp[MF_D:, :], preferred_element_type=jnp.float32)
    o_r[...] = p + bp_r[...]


def _tc_mlp(mf, mlpu, mlpi, W0, b0, W1, b1, W2, b2, Wp, bp):
    BLK = 2048
    grid = (B // BLK,)

    def full(shape):
        return pl.BlockSpec(shape, lambda i: (0,) * len(shape))

    return pl.pallas_call(
        _tc_body,
        grid=grid,
        in_specs=[
            pl.BlockSpec((BLK, MF_D), lambda i: (i, 0)),
            pl.BlockSpec((BLK, 16), lambda i: (i, 0)),
            pl.BlockSpec((BLK, 16), lambda i: (i, 0)),
            full(W0.shape), full(b0.shape), full(W1.shape), full(b1.shape),
            full(W2.shape), full(b2.shape), full(Wp.shape), full(bp.shape),
        ],
        out_specs=pl.BlockSpec((BLK, 1), lambda i: (i, 0)),
        out_shape=jax.ShapeDtypeStruct((B, 1), jnp.float32),
    )(mf, mlpu, mlpi, W0, b0, W1, b1, W2, b2, Wp, bp)


def kernel(user, item, mf_emb_user, mf_emb_item, mlp_emb_user, mlp_emb_item,
           W0, b0, W1, b1, W2, b2, Wp, bp):
    user = user.astype(jnp.int32)
    item = item.astype(jnp.int32)
    u2 = user.reshape(NW, BPW)
    i2 = item.reshape(NW, BPW)
    mf, mlpu, mlpi = _sc_gather(
        u2, i2,
        mf_emb_user.reshape(-1, W), mf_emb_item.reshape(-1, W),
        mlp_emb_user, mlp_emb_item)
    mf = mf.reshape(B, MF_D)
    mlpu = mlpu.reshape(B, 16)
    mlpi = mlpi.reshape(B, 16)
    return _tc_mlp(
        mf, mlpu, mlpi,
        W0, b0.reshape(1, -1), W1, b1.reshape(1, -1),
        W2, b2.reshape(1, -1), Wp, bp.reshape(1, 1))
